# baseline (speedup 1.0000x reference)
import jax
import jax.numpy as jnp
from jax.experimental import pallas as pl
from jax.experimental.pallas import tpu as pltpu

_CHUNKS = (2500, 2500, 5000) + (10000,) * 9
_OFFS = tuple(sum(_CHUNKS[:k]) for k in range(len(_CHUNKS)))
_NBUF = 4
_MAXC = max(_CHUNKS)


def _gcn_body(adjt_hbm, x_hbm, w_ref, b_ref, o_ref, a_buf, x_buf, sem):
    def _copies(k):
        slot = k % _NBUF
        c = _CHUNKS[k]
        o = _OFFS[k]
        return (
            pltpu.make_async_copy(
                adjt_hbm.at[pl.ds(o, c), :],
                a_buf.at[slot, pl.ds(0, c), :],
                sem.at[0, slot],
            ),
            pltpu.make_async_copy(
                x_hbm.at[pl.ds(o, c), :],
                x_buf.at[slot, pl.ds(0, c), :],
                sem.at[1, slot],
            ),
        )

    for k in range(_NBUF):
        for cp in _copies(k):
            cp.start()

    acc = jnp.zeros(o_ref.shape, jnp.float32)
    for k in range(len(_CHUNKS)):
        for cp in _copies(k):
            cp.wait()
        slot = k % _NBUF
        c = _CHUNKS[k]
        acc = acc + jax.lax.dot_general(
            a_buf[slot, pl.ds(0, c), :],
            x_buf[slot, pl.ds(0, c), :],
            dimension_numbers=(((0,), (0,)), ((), ())),
            preferred_element_type=jnp.float32,
        )
        if k + _NBUF < len(_CHUNKS):
            for cp in _copies(k + _NBUF):
                cp.start()

    o_ref[...] = (
        jnp.dot(acc, w_ref[...], preferred_element_type=jnp.float32) + b_ref[...]
    )


@jax.jit
def kernel(x, adj, weight, bias):
    n, f_in = x.shape
    f_out = adj.shape[0]
    adjt = jnp.swapaxes(adj, 0, 1)
    bias2 = bias.reshape(1, f_out)
    return pl.pallas_call(
        _gcn_body,
        in_specs=[
            pl.BlockSpec(memory_space=pltpu.MemorySpace.HBM),
            pl.BlockSpec(memory_space=pltpu.MemorySpace.HBM),
            pl.BlockSpec((f_in, f_out), lambda: (0, 0)),
            pl.BlockSpec((1, f_out), lambda: (0, 0)),
        ],
        out_specs=pl.BlockSpec((f_out, f_out), lambda: (0, 0)),
        out_shape=jax.ShapeDtypeStruct((f_out, f_out), jnp.float32),
        scratch_shapes=[
            pltpu.VMEM((_NBUF, _MAXC, f_out), jnp.float32),
            pltpu.VMEM((_NBUF, _MAXC, f_in), jnp.float32),
            pltpu.SemaphoreType.DMA((2, _NBUF)),
        ],
        compiler_params=pltpu.CompilerParams(
            vmem_limit_bytes=50 * 1024 * 1024,
        ),
    )(adjt, x, weight, bias2)


# speedup vs baseline: 1.0200x; 1.0200x over previous
import jax
import jax.numpy as jnp
from jax.experimental import pallas as pl
from jax.experimental.pallas import tpu as pltpu

_CHUNKS = (2500, 2500) + (5000,) * 19
_OFFS = tuple(sum(_CHUNKS[:k]) for k in range(len(_CHUNKS)))
_NBUF = 6
_MAXC = max(_CHUNKS)


def _gcn_body(adjt_hbm, x_hbm, w_ref, b_ref, o_ref, a_buf, x_buf, sem):
    def _copies(k):
        slot = k % _NBUF
        c = _CHUNKS[k]
        o = _OFFS[k]
        return (
            pltpu.make_async_copy(
                adjt_hbm.at[pl.ds(o, c), :],
                a_buf.at[slot, pl.ds(0, c), :],
                sem.at[0, slot],
            ),
            pltpu.make_async_copy(
                x_hbm.at[pl.ds(o, c), :],
                x_buf.at[slot, pl.ds(0, c), :],
                sem.at[1, slot],
            ),
        )

    for k in range(_NBUF):
        for cp in _copies(k):
            cp.start()

    acc = jnp.zeros(o_ref.shape, jnp.float32)
    for k in range(len(_CHUNKS)):
        for cp in _copies(k):
            cp.wait()
        slot = k % _NBUF
        c = _CHUNKS[k]
        acc = acc + jax.lax.dot_general(
            a_buf[slot, pl.ds(0, c), :],
            x_buf[slot, pl.ds(0, c), :],
            dimension_numbers=(((0,), (0,)), ((), ())),
            preferred_element_type=jnp.float32,
        )
        if k + _NBUF < len(_CHUNKS):
            for cp in _copies(k + _NBUF):
                cp.start()

    o_ref[...] = (
        jnp.dot(acc, w_ref[...], preferred_element_type=jnp.float32) + b_ref[...]
    )


@jax.jit
def kernel(x, adj, weight, bias):
    n, f_in = x.shape
    f_out = adj.shape[0]
    adjt = jnp.swapaxes(adj, 0, 1)
    bias2 = bias.reshape(1, f_out)
    return pl.pallas_call(
        _gcn_body,
        in_specs=[
            pl.BlockSpec(memory_space=pltpu.MemorySpace.HBM),
            pl.BlockSpec(memory_space=pltpu.MemorySpace.HBM),
            pl.BlockSpec((f_in, f_out), lambda: (0, 0)),
            pl.BlockSpec((1, f_out), lambda: (0, 0)),
        ],
        out_specs=pl.BlockSpec((f_out, f_out), lambda: (0, 0)),
        out_shape=jax.ShapeDtypeStruct((f_out, f_out), jnp.float32),
        scratch_shapes=[
            pltpu.VMEM((_NBUF, _MAXC, f_out), jnp.float32),
            pltpu.VMEM((_NBUF, _MAXC, f_in), jnp.float32),
            pltpu.SemaphoreType.DMA((2, _NBUF)),
        ],
        compiler_params=pltpu.CompilerParams(
            vmem_limit_bytes=50 * 1024 * 1024,
        ),
    )(adjt, x, weight, bias2)


# speedup vs baseline: 1.0853x; 1.0640x over previous
import functools

import jax
import jax.numpy as jnp
from jax.experimental import pallas as pl
from jax.experimental.pallas import tpu as pltpu

_TILE = 5000


def _gcn_body(a1_ref, x1_ref, a2_ref, x2_ref, w_ref, b_ref, o_ref, acc_ref):
    i = pl.program_id(0)
    nt = pl.num_programs(0)

    @pl.when(i == 0)
    def _init():
        acc_ref[...] = jnp.zeros_like(acc_ref)

    dn = (((0,), (0,)), ((), ()))
    acc_ref[...] += jax.lax.dot_general(
        a1_ref[...], x1_ref[...], dn, preferred_element_type=jnp.float32
    ) + jax.lax.dot_general(
        a2_ref[...], x2_ref[...], dn, preferred_element_type=jnp.float32
    )

    @pl.when(i == nt - 1)
    def _finish():
        o_ref[...] = (
            jnp.dot(acc_ref[...], w_ref[...], preferred_element_type=jnp.float32)
            + b_ref[...]
        )


@jax.jit
def kernel(x, adj, weight, bias):
    n, f_in = x.shape
    f_out = adj.shape[0]
    tile = _TILE
    nt = n // (2 * tile)
    adjt = jnp.swapaxes(adj, 0, 1)
    bias2 = bias.reshape(1, f_out)
    half = nt
    return pl.pallas_call(
        _gcn_body,
        grid=(nt,),
        in_specs=[
            pl.BlockSpec((tile, f_out), lambda i: (i, 0)),
            pl.BlockSpec((tile, f_in), lambda i: (i, 0)),
            pl.BlockSpec((tile, f_out), lambda i: (i + half, 0)),
            pl.BlockSpec((tile, f_in), lambda i: (i + half, 0)),
            pl.BlockSpec((f_in, f_out), lambda i: (0, 0)),
            pl.BlockSpec((1, f_out), lambda i: (0, 0)),
        ],
        out_specs=pl.BlockSpec((f_out, f_out), lambda i: (0, 0)),
        out_shape=jax.ShapeDtypeStruct((f_out, f_out), jnp.float32),
        scratch_shapes=[pltpu.VMEM((f_out, f_out), jnp.float32)],
        compiler_params=pltpu.CompilerParams(
            dimension_semantics=("arbitrary",),
        ),
    )(adjt, x, adjt, x, weight, bias2)
